# baseline (speedup 1.0000x reference)
import functools

import jax
import jax.numpy as jnp
from jax import lax
from jax.experimental import pallas as pl
from jax.experimental.pallas import tpu as pltpu
from jax.experimental.pallas import tpu_sc as plsc

NUM_USERS = 100000
D = 64
P = 8
B = 16384
SLAB = P * D

NC = 2
NS = 16
NW = NC * NS
RPW = B // NW
CHUNK = 16
NCHUNK = RPW // CHUNK
NBUF = 4


def _z_body(x_ref, wt_ref, b_ref, z_ref):
    h = jnp.dot(x_ref[...], wt_ref[...], preferred_element_type=jnp.float32)
    h = h + b_ref[...]
    s2 = jnp.sum(h * h, axis=1, keepdims=True)
    z_ref[...] = h * lax.rsqrt(jnp.maximum(s2, 1e-24))


def _compute_z(x, wt, b2d):
    blk = 2048
    return pl.pallas_call(
        _z_body,
        grid=(B // blk,),
        in_specs=[
            pl.BlockSpec((blk, D), lambda i: (i, 0)),
            pl.BlockSpec((D, D), lambda i: (0, 0)),
            pl.BlockSpec((1, D), lambda i: (0, 0)),
        ],
        out_specs=pl.BlockSpec((blk, D), lambda i: (i, 0)),
        out_shape=jax.ShapeDtypeStruct((B, D), jnp.float32),
    )(x, wt, b2d)


def _rsqrt16(s):
    i = plsc.bitcast(s, jnp.int32)
    i = jnp.int32(0x5F3759DF) - (i >> 1)
    y = plsc.bitcast(i, jnp.float32)
    for _ in range(3):
        y = y * (1.5 - 0.5 * s * y * y)
    return y


def _sc_body(idx_hbm, z_hbm, proto_hbm, out_hbm,
             idx_v, z_v, out_v, slabs, sems):
    wid = lax.axis_index("s") * NC + lax.axis_index("c")
    base = wid * RPW
    ii = lax.iota(jnp.int32, 16)

    with jax.named_scope("idx_stage"):
        pltpu.sync_copy(idx_hbm.at[pl.ds(base, RPW)], idx_v)

    def gather(c, which):
        pltpu.make_async_copy(
            proto_hbm.at[idx_v.at[pl.ds(c * CHUNK, CHUNK)]],
            slabs[which], sems[which],
        ).start()

    def drain(which):
        pltpu.make_async_copy(
            proto_hbm.at[pl.ds(0, CHUNK)], slabs[which], sems[which]
        ).wait()

    for c in range(NBUF):
        gather(c, c)

    with jax.named_scope("z_stage"):
        pltpu.sync_copy(z_hbm.at[pl.ds(base, RPW)], z_v)

    def chunk_work(c, which):
        drain(which)
        slab = slabs[which]
        rz = (c * CHUNK) + ii
        zero = jnp.zeros((16,), jnp.float32)
        acc0 = ((zero,) * P, (zero,) * P)

        def dbody(k, acc, rz=rz, slab=slab):
            s2a, dpa = acc
            s2n, dpn = list(s2a), list(dpa)
            for u in range(4):
                cd = jnp.full((16,), k * 4, jnp.int32) + u
                za = plsc.load_gather(z_v, [rz, cd])
                for pp in range(P):
                    a = plsc.load_gather(slab, [ii, cd + (pp * D)])
                    s2n[pp] = s2n[pp] + a * a
                    dpn[pp] = dpn[pp] + za * a
            return (tuple(s2n), tuple(dpn))

        s2a, dpa = plsc.parallel_loop(0, D // 4, 1, unroll=2, carry=acc0)(
            lambda k, acc: dbody(k, acc))

        score = jnp.zeros((16,), jnp.float32)
        for pp in range(P):
            score = score + dpa[pp] * _rsqrt16(jnp.maximum(s2a[pp], 1e-24))
        out_v[pl.ds(c * CHUNK, 16)] = score

        cn = c + NBUF
        @pl.when(cn < NCHUNK)
        def _():
            gather(cn, which)

    def group(k, carry):
        for b in range(NBUF):
            chunk_work(k * NBUF + b, b)
        return carry

    with jax.named_scope("mainloop"):
        lax.fori_loop(0, NCHUNK // NBUF, group, 0)

    pltpu.sync_copy(out_v, out_hbm.at[pl.ds(base, RPW)])


@functools.partial(
    pl.kernel,
    mesh=plsc.VectorSubcoreMesh(core_axis_name="c", subcore_axis_name="s"),
    out_type=jax.ShapeDtypeStruct((B,), jnp.float32),
    compiler_params=pltpu.CompilerParams(
        use_tc_tiling_on_sc=False, needs_layout_passes=False,
        disable_bounds_checks=True, disable_semaphore_checks=True),
    scratch_types=(
        [
            pltpu.VMEM((RPW,), jnp.int32),
            pltpu.VMEM((RPW, D), jnp.float32),
            pltpu.VMEM((RPW,), jnp.float32),
        ]
        + [pltpu.VMEM((CHUNK, SLAB), jnp.float32) for _ in range(NBUF)]
        + [pltpu.SemaphoreType.DMA for _ in range(NBUF)]
    ),
)
def _sc_score(idx_hbm, z_hbm, proto_hbm, out_hbm,
              idx_v, z_v, out_v, *rest):
    _sc_body(idx_hbm, z_hbm, proto_hbm, out_hbm,
             idx_v, z_v, out_v, list(rest[:NBUF]), list(rest[NBUF:]))


def kernel(uidx, x, out_proto, W, b):
    z = _compute_z(x, W.T, b.reshape(1, D))
    proto2d = out_proto.reshape(NUM_USERS, SLAB)
    score = _sc_score(uidx.astype(jnp.int32), z, proto2d)
    return score.reshape(B, 1)
